# Initial kernel scaffold; baseline (speedup 1.0000x reference)
import jax
import jax.numpy as jnp
from jax.experimental import pallas as pl


def kernel(x, edge_index, W0, b0, W1, b1, W2, b2, Wb, bb):
    raise NotImplementedError("write your pallas kernel here")


# speedup vs baseline: 8.4441x; 8.4441x over previous
import functools
import jax
import jax.numpy as jnp
from jax import lax
from jax.experimental import pallas as pl
from jax.experimental.pallas import tpu as pltpu, tpu_sc as plsc

N = 10000
D = 128
F = 64
E = 320000
NC = 2
NS = 16
NW = NC * NS
EPW = E // NW
CH = 128
NFULL = EPW // CH
TAIL = EPW - NFULL * CH

_mesh = functools.partial(
    plsc.VectorSubcoreMesh, core_axis_name="c", subcore_axis_name="s")


def _wid():
    return lax.axis_index("c") * NS + lax.axis_index("s")


@functools.partial(
    pl.kernel,
    out_type=jax.ShapeDtypeStruct((NW, N), jnp.float32),
    mesh=_mesh(),
    compiler_params=pltpu.CompilerParams(needs_layout_passes=False, use_tc_tiling_on_sc=False),
    scratch_types=[
        pltpu.VMEM((EPW,), jnp.int32),
        pltpu.VMEM((N,), jnp.float32),
        pltpu.SemaphoreType.DMA,
    ],
)
def _sc_deg(dst_hbm, out_hbm, dstv, degv, sem):
    wid = _wid()
    base = wid * EPW

    zero16 = jnp.zeros((16,), jnp.float32)

    def zbody(i, _):
        degv[pl.ds(i * 16, 16)] = zero16
        return 0
    lax.fori_loop(0, N // 16, zbody, 0)

    pltpu.sync_copy(dst_hbm.at[pl.ds(base, EPW)], dstv)

    one16 = jnp.ones((16,), jnp.float32)

    def body(i, _):
        idx = dstv[pl.ds(i * 16, 16)]
        plsc.addupdate_scatter(degv, [idx], one16)
        return 0
    lax.fori_loop(0, EPW // 16, body, 0)

    pltpu.sync_copy(degv, out_hbm.at[wid])


@functools.partial(
    pl.kernel,
    out_type=jax.ShapeDtypeStruct((NC, N, F), jnp.float32),
    mesh=_mesh(),
    compiler_params=pltpu.CompilerParams(needs_layout_passes=False, use_tc_tiling_on_sc=False),
    scratch_types=[
        pltpu.VMEM((CH,), jnp.int32),
        pltpu.VMEM((CH,), jnp.int32),
        pltpu.VMEM((CH, F), jnp.float32),
        pltpu.VMEM((TAIL,), jnp.int32),
        pltpu.VMEM((TAIL,), jnp.int32),
        pltpu.VMEM((TAIL, F), jnp.float32),
        pltpu.VMEM_SHARED((N, F), jnp.float32),
        pltpu.SemaphoreType.DMA,
    ],
)
def _sc_prop(tp_hbm, src_hbm, dst_hbm, zeros_hbm, out_hbm,
             idxg, idxs, rows, idxgt, idxst, rowst, acc, sem):
    c = lax.axis_index("c")
    s = lax.axis_index("s")
    base = _wid() * EPW

    @pl.when(s < 10)
    def _():
        pltpu.sync_copy(zeros_hbm.at[pl.ds(s * 1000, 1000)],
                        acc.at[pl.ds(s * 1000, 1000)])
    plsc.subcore_barrier()

    def chunk(j, _):
        off = base + j * CH
        pltpu.sync_copy(src_hbm.at[pl.ds(off, CH)], idxg)
        pltpu.sync_copy(dst_hbm.at[pl.ds(off, CH)], idxs)
        pltpu.async_copy(tp_hbm.at[idxg], rows, sem).wait()
        pltpu.sync_copy(rows, acc.at[idxs], add=True)
        return 0
    lax.fori_loop(0, NFULL, chunk, 0)

    off = base + NFULL * CH
    pltpu.sync_copy(src_hbm.at[pl.ds(off, TAIL)], idxgt)
    pltpu.sync_copy(dst_hbm.at[pl.ds(off, TAIL)], idxst)
    pltpu.async_copy(tp_hbm.at[idxgt], rowst, sem).wait()
    pltpu.sync_copy(rowst, acc.at[idxst], add=True)

    plsc.subcore_barrier()

    @pl.when(s < 10)
    def _():
        pltpu.sync_copy(acc.at[pl.ds(s * 1000, 1000)],
                        out_hbm.at[c, pl.ds(s * 1000, 1000)])


@functools.partial(
    pl.kernel,
    out_type=jax.ShapeDtypeStruct((E,), jnp.float32),
    mesh=_mesh(),
    compiler_params=pltpu.CompilerParams(needs_layout_passes=False, use_tc_tiling_on_sc=False),
    scratch_types=[
        pltpu.VMEM((CH,), jnp.int32),
        pltpu.VMEM((CH,), jnp.int32),
        pltpu.VMEM((CH, F), jnp.float32),
        pltpu.VMEM((CH, F), jnp.float32),
        pltpu.VMEM((EPW,), jnp.float32),
        pltpu.VMEM((16,), jnp.float32),
        pltpu.SemaphoreType.DMA,
    ],
)
def _sc_score(u_hbm, h_hbm, src_hbm, dst_hbm, bb_hbm, out_hbm,
              idxg, idxs, urows, hrows, scores, bbv, sem):
    base = _wid() * EPW
    pltpu.sync_copy(bb_hbm, bbv)
    bbvec = bbv[...]

    iota16 = lax.iota(jnp.int32, 16)

    def do_chunk(j, nedge):
        off = base + j * CH
        pltpu.sync_copy(src_hbm.at[pl.ds(off, nedge)], idxg.at[pl.ds(0, nedge)])
        pltpu.sync_copy(dst_hbm.at[pl.ds(off, nedge)], idxs.at[pl.ds(0, nedge)])
        pltpu.async_copy(u_hbm.at[idxg], urows, sem).wait()
        pltpu.async_copy(h_hbm.at[idxs], hrows, sem).wait()

        def group(g, _):
            rowi = g * 16 + iota16
            acc = jnp.zeros((16,), jnp.float32)
            for f in range(F):
                coli = jnp.full((16,), f, jnp.int32)
                acc = acc + (plsc.load_gather(urows, [rowi, coli])
                             * plsc.load_gather(hrows, [rowi, coli]))
            scores[pl.ds(j * CH + g * 16, 16)] = acc + bbvec
            return 0
        lax.fori_loop(0, nedge // 16, group, 0)

    def chunk(j, _):
        do_chunk(j, CH)
        return 0
    lax.fori_loop(0, NFULL, chunk, 0)
    do_chunk(NFULL, TAIL)

    pltpu.sync_copy(scores, out_hbm.at[pl.ds(base, EPW)])


def _tc1_body(x_ref, w0_ref, b0_ref, w1_ref, deg_ref, t1p_ref, dinv_ref):
    h0 = jnp.dot(x_ref[...], w0_ref[...],
                 preferred_element_type=jnp.float32) + b0_ref[...]
    t1 = jnp.dot(h0, w1_ref[...], preferred_element_type=jnp.float32)
    deg = jnp.sum(deg_ref[...], axis=1, keepdims=True) + 1.0
    dinv = lax.rsqrt(deg)
    dinv_ref[...] = dinv
    t1p_ref[...] = t1 * dinv


def _tc1(x, W0, b0, W1, deg3):
    return pl.pallas_call(
        _tc1_body,
        out_shape=[jax.ShapeDtypeStruct((N, F), jnp.float32),
                   jax.ShapeDtypeStruct((N, 1), jnp.float32)],
    )(x, W0, b0, W1, deg3)


def _tc2_body(pa_ref, pb_ref, tp_ref, dinv_ref, b1_ref, w2_ref, out_ref):
    dinv = dinv_ref[...]
    ssum = pa_ref[...] + pb_ref[...] + tp_ref[...]
    h1 = jnp.maximum(ssum * dinv + b1_ref[...], 0.0)
    t2 = jnp.dot(h1, w2_ref[...], preferred_element_type=jnp.float32)
    out_ref[...] = t2 * dinv


def _tc2(pa, pb, tp, dinv, b1, W2):
    return pl.pallas_call(
        _tc2_body,
        out_shape=jax.ShapeDtypeStruct((N, F), jnp.float32),
    )(pa, pb, tp, dinv, b1, W2)


def _tc3_body(pa_ref, pb_ref, tp_ref, dinv_ref, b2_ref, wb_ref,
              h2_ref, u_ref):
    ssum = pa_ref[...] + pb_ref[...] + tp_ref[...]
    h2 = ssum * dinv_ref[...] + b2_ref[...]
    h2_ref[...] = h2
    u_ref[...] = jnp.dot(h2, wb_ref[...], preferred_element_type=jnp.float32)


def _tc3(pa, pb, tp, dinv, b2, Wb0):
    return pl.pallas_call(
        _tc3_body,
        out_shape=[jax.ShapeDtypeStruct((N, F), jnp.float32),
                   jax.ShapeDtypeStruct((N, F), jnp.float32)],
    )(pa, pb, tp, dinv, b2, Wb0)


def kernel(x, edge_index, W0, b0, W1, b1, W2, b2, Wb, bb):
    src = edge_index[0]
    dst = edge_index[1]

    degp = _sc_deg(dst)
    t1p, dinv = _tc1(x, W0, b0, W1, degp.T)

    zeros = jnp.zeros((N, F), jnp.float32)
    p1 = _sc_prop(t1p, src, dst, zeros)
    t2p = _tc2(p1[0], p1[1], t1p, dinv, b1, W2)
    p2 = _sc_prop(t2p, src, dst, zeros)
    h2, u = _tc3(p2[0], p2[1], t2p, dinv, b2, Wb[0])

    bb16 = jnp.full((16,), bb[0], jnp.float32)
    return _sc_score(u, h2, src, dst, bb16)
